# Initial kernel scaffold; baseline (speedup 1.0000x reference)
import jax
import jax.numpy as jnp
from jax.experimental import pallas as pl


def kernel(x, edge_index, W1, b1, W2, b2):
    raise NotImplementedError("write your pallas kernel here")


# speedup vs baseline: 20.6620x; 20.6620x over previous
import functools

import jax
import jax.numpy as jnp
from jax import lax
from jax.experimental import pallas as pl
from jax.experimental.pallas import tpu as pltpu
from jax.experimental.pallas import tpu_sc as plsc

N_NODES = 10000
N_EDGES = 320000
D = 128
NC = 2
NS = 16
NW = NC * NS
EPT = N_EDGES // NW
CHUNK = 80
NCHUNK = EPT // CHUNK
ROWS_PT = 640
_LAST_R0 = N_NODES - ROWS_PT
DRAIN_C = ROWS_PT // CHUNK

_MESH = plsc.VectorSubcoreMesh(
    core_axis_name="c", subcore_axis_name="s", num_cores=NC, num_subcores=NS
)


@functools.partial(
    pl.kernel,
    out_type=jax.ShapeDtypeStruct((NC * N_NODES,), jnp.float32),
    mesh=_MESH,
    scratch_types=[
        pltpu.VMEM((NCHUNK, CHUNK), jnp.int32),
        pltpu.VMEM((CHUNK,), jnp.float32),
        pltpu.VMEM((ROWS_PT,), jnp.float32),
        pltpu.VMEM_SHARED((N_NODES,), jnp.float32),
    ],
)
def _deg_kernel(dst_hbm, out_hbm, didx, ones_v, zbuf, acc_sh):
    cid = lax.axis_index("c")
    sid = lax.axis_index("s")
    wid = cid * NS + sid
    r0 = jnp.minimum(sid * ROWS_PT, _LAST_R0)

    @pl.loop(0, ROWS_PT, step=16)
    def _(i):
        zbuf[pl.ds(i, 16)] = jnp.zeros((16,), jnp.float32)

    @pl.loop(0, CHUNK, step=16)
    def _(i):
        ones_v[pl.ds(i, 16)] = jnp.full((16,), 1.0, jnp.float32)

    pltpu.sync_copy(zbuf, acc_sh.at[pl.ds(r0, ROWS_PT)])
    pltpu.sync_copy(dst_hbm.at[wid], didx)
    plsc.subcore_barrier()

    @pl.loop(0, NCHUNK)
    def _(j):
        pltpu.sync_copy(ones_v, acc_sh.at[didx.at[j]], add=True)

    plsc.subcore_barrier()
    pltpu.sync_copy(acc_sh.at[pl.ds(r0, ROWS_PT)], zbuf)
    pltpu.sync_copy(zbuf, out_hbm.at[pl.ds(cid * N_NODES + r0, ROWS_PT)])


@functools.partial(
    pl.kernel,
    out_type=jax.ShapeDtypeStruct((NC, N_NODES, D), jnp.float32),
    mesh=_MESH,
    scratch_types=[
        pltpu.VMEM((NCHUNK, CHUNK), jnp.int32),
        pltpu.VMEM((NCHUNK, CHUNK), jnp.int32),
        pltpu.VMEM((CHUNK, D), jnp.float32),
        pltpu.VMEM_SHARED((N_NODES, D), jnp.float32),
        pltpu.SemaphoreType.DMA,
    ],
)
def _agg_kernel(table_hbm, src_hbm, dst_hbm, out_hbm,
                sidx, didx, rows, acc_sh, sem):
    cid = lax.axis_index("c")
    sid = lax.axis_index("s")
    wid = cid * NS + sid
    r0 = jnp.minimum(sid * ROWS_PT, _LAST_R0)

    @pl.loop(0, CHUNK)
    def _(i):
        @pl.loop(0, D, step=16)
        def _(j):
            rows[i, pl.ds(j, 16)] = jnp.zeros((16,), jnp.float32)

    @pl.loop(0, DRAIN_C)
    def _(k):
        pltpu.sync_copy(rows, acc_sh.at[pl.ds(r0 + k * CHUNK, CHUNK)])

    pltpu.sync_copy(src_hbm.at[wid], sidx)
    pltpu.sync_copy(dst_hbm.at[wid], didx)
    plsc.subcore_barrier()

    @pl.loop(0, NCHUNK)
    def _(j):
        pltpu.async_copy(table_hbm.at[sidx.at[j]], rows, sem).wait()
        pltpu.sync_copy(rows, acc_sh.at[didx.at[j]], add=True)

    plsc.subcore_barrier()

    @pl.loop(0, DRAIN_C)
    def _(k):
        rr = r0 + k * CHUNK
        pltpu.sync_copy(acc_sh.at[pl.ds(rr, CHUNK)], rows)
        pltpu.sync_copy(rows, out_hbm.at[cid, pl.ds(rr, CHUNK)])


_RB = 1000
_GRID = N_NODES // _RB

_HIGH = jax.lax.Precision.HIGHEST
_DN = (((1,), (0,)), ((), ()))


def _row_spec(width):
    return pl.BlockSpec((_RB, width), lambda i: (i, 0))


def _full_spec(shape):
    return pl.BlockSpec(shape, lambda i: tuple(0 for _ in shape))


def _prescale_body(x_ref, dinv_ref, xs_ref):
    xs_ref[...] = x_ref[...] * dinv_ref[...]


_prescale = pl.pallas_call(
    _prescale_body,
    grid=(_GRID,),
    in_specs=[_row_spec(D), _row_spec(1)],
    out_specs=_row_spec(D),
    out_shape=jax.ShapeDtypeStruct((N_NODES, D), jnp.float32),
)


def _mid_body(p0_ref, p1_ref, x_ref, dinv_ref, dinv2_ref,
              w1_ref, b1_ref, w2_ref, y_ref, ys_ref):
    ax = (dinv_ref[...] * (p0_ref[...] + p1_ref[...])
          + dinv2_ref[...] * x_ref[...])
    h1 = lax.dot_general(ax, w1_ref[...], _DN, precision=_HIGH,
                         preferred_element_type=jnp.float32)
    h1 = jnp.maximum(h1 + b1_ref[...], 0.0)
    y = lax.dot_general(h1, w2_ref[...], _DN, precision=_HIGH,
                        preferred_element_type=jnp.float32)
    y_ref[...] = y
    ys_ref[...] = y * dinv_ref[...]


_mid = pl.pallas_call(
    _mid_body,
    grid=(_GRID,),
    in_specs=[_row_spec(D), _row_spec(D), _row_spec(D), _row_spec(1),
              _row_spec(1), _full_spec((D, 2 * D)), _full_spec((1, 2 * D)),
              _full_spec((2 * D, D))],
    out_specs=[_row_spec(D), _row_spec(D)],
    out_shape=[jax.ShapeDtypeStruct((N_NODES, D), jnp.float32),
               jax.ShapeDtypeStruct((N_NODES, D), jnp.float32)],
)


def _final_body(q0_ref, q1_ref, y_ref, dinv_ref, dinv2_ref, b2_ref, out_ref):
    agg = dinv_ref[...] * (q0_ref[...] + q1_ref[...])
    out_ref[...] = jnp.maximum(agg + dinv2_ref[...] * y_ref[...]
                               + b2_ref[...], 0.0)


_final = pl.pallas_call(
    _final_body,
    grid=(_GRID,),
    in_specs=[_row_spec(D), _row_spec(D), _row_spec(D), _row_spec(1),
              _row_spec(1), _full_spec((1, D))],
    out_specs=_row_spec(D),
    out_shape=jax.ShapeDtypeStruct((N_NODES, D), jnp.float32),
)


def kernel(x, edge_index, W1, b1, W2, b2):
    ei = edge_index.astype(jnp.int32)
    src3 = ei[0].reshape(NW, NCHUNK, CHUNK)
    dst3 = ei[1].reshape(NW, NCHUNK, CHUNK)
    deg_p = _deg_kernel(dst3).reshape(NC, N_NODES)
    deg = deg_p[0] + deg_p[1] + 1.0
    dinv = lax.rsqrt(deg).reshape(N_NODES, 1)
    dinv2 = (1.0 / deg).reshape(N_NODES, 1)

    xs = _prescale(x, dinv)
    agg1 = _agg_kernel(xs, src3, dst3)
    y, ys = _mid(agg1[0], agg1[1], x, dinv, dinv2,
                 W1, b1.reshape(1, 2 * D), W2)
    agg2 = _agg_kernel(ys, src3, dst3)
    return _final(agg2[0], agg2[1], y, dinv, dinv2, b2.reshape(1, D))


# speedup vs baseline: 25.4141x; 1.2300x over previous
import functools

import jax
import jax.numpy as jnp
from jax import lax
from jax.experimental import pallas as pl
from jax.experimental.pallas import tpu as pltpu
from jax.experimental.pallas import tpu_sc as plsc

N_NODES = 10000
N_EDGES = 320000
D = 128
NC = 2
NS = 16
NW = NC * NS
EPT = N_EDGES // NW
CHUNK = 80
NCHUNK = EPT // CHUNK
ROWS_PT = 640
_LAST_R0 = N_NODES - ROWS_PT
ACH = 80
ANCH = EPT // ACH
DRAIN_C = ROWS_PT // ACH
_DSLAB = 25
_NSLAB = NCHUNK // _DSLAB

_MESH = plsc.VectorSubcoreMesh(
    core_axis_name="c", subcore_axis_name="s", num_cores=NC, num_subcores=NS
)


@functools.partial(
    pl.kernel,
    out_type=jax.ShapeDtypeStruct((NC * N_NODES,), jnp.float32),
    mesh=_MESH,
    scratch_types=[
        pltpu.VMEM((_DSLAB, CHUNK), jnp.int32),
        pltpu.VMEM((CHUNK,), jnp.float32),
        pltpu.VMEM((ROWS_PT,), jnp.float32),
        pltpu.VMEM_SHARED((N_NODES,), jnp.float32),
    ],
)
def _deg_kernel(dst_hbm, out_hbm, didx, ones_v, zbuf, acc_sh):
    cid = lax.axis_index("c")
    sid = lax.axis_index("s")
    wid = cid * NS + sid
    r0 = jnp.minimum(sid * ROWS_PT, _LAST_R0)

    @pl.loop(0, ROWS_PT, step=16)
    def _(i):
        zbuf[pl.ds(i, 16)] = jnp.zeros((16,), jnp.float32)

    @pl.loop(0, CHUNK, step=16)
    def _(i):
        ones_v[pl.ds(i, 16)] = jnp.full((16,), 1.0, jnp.float32)

    pltpu.sync_copy(zbuf, acc_sh.at[pl.ds(r0, ROWS_PT)])
    plsc.subcore_barrier()

    @pl.loop(0, _NSLAB)
    def _(s):
        pltpu.sync_copy(dst_hbm.at[wid * _NSLAB + s], didx)

        @pl.loop(0, _DSLAB)
        def _(j):
            pltpu.sync_copy(ones_v, acc_sh.at[didx.at[j]], add=True)

    plsc.subcore_barrier()
    pltpu.sync_copy(acc_sh.at[pl.ds(r0, ROWS_PT)], zbuf)
    pltpu.sync_copy(zbuf, out_hbm.at[pl.ds(cid * N_NODES + r0, ROWS_PT)])


@functools.partial(
    pl.kernel,
    out_type=jax.ShapeDtypeStruct((NC, N_NODES, D), jnp.float32),
    mesh=_MESH,
    scratch_types=[
        pltpu.VMEM((EPT,), jnp.int32),
        pltpu.VMEM((EPT,), jnp.int32),
        pltpu.VMEM((2, ACH, D), jnp.float32),
        pltpu.VMEM_SHARED((N_NODES, D), jnp.float32),
    ] + [pltpu.SemaphoreType.DMA] * 4,
)
def _agg_kernel(table_hbm, src_hbm, dst_hbm, out_hbm,
                sidx, didx, rowbuf, acc_sh, g0, g1, s0, s1):
    rows = (rowbuf.at[0], rowbuf.at[1])
    gsem = (g0, g1)
    ssem = (s0, s1)
    cid = lax.axis_index("c")
    sid = lax.axis_index("s")
    wid = cid * NS + sid
    r0 = jnp.minimum(sid * ROWS_PT, _LAST_R0)

    @pl.loop(0, ACH)
    def _(i):
        @pl.loop(0, D, step=16)
        def _(j):
            rowbuf[0, i, pl.ds(j, 16)] = jnp.zeros((16,), jnp.float32)

    @pl.loop(0, DRAIN_C)
    def _(k):
        pltpu.sync_copy(rows[0], acc_sh.at[pl.ds(r0 + k * ACH, ACH)])

    pltpu.sync_copy(src_hbm.at[wid], sidx)
    pltpu.sync_copy(dst_hbm.at[wid], didx)
    plsc.subcore_barrier()

    def _gather(j, b):
        pltpu.async_copy(table_hbm.at[sidx.at[pl.ds(j * ACH, ACH)]],
                         rows[b], gsem[b])

    def _wait_gather(j, b):
        pltpu.make_async_copy(table_hbm.at[sidx.at[pl.ds(j * ACH, ACH)]],
                              rows[b], gsem[b]).wait()

    def _scatter(j, b):
        pltpu.async_copy(rows[b], acc_sh.at[didx.at[pl.ds(j * ACH, ACH)]],
                         ssem[b], add=True)

    def _wait_scatter(j, b):
        pltpu.make_async_copy(rows[b], acc_sh.at[didx.at[pl.ds(j * ACH, ACH)]],
                              ssem[b]).wait()

    def _step(j, b, wait_scat=True, issue_gather=True):
        _wait_gather(j, b)
        _scatter(j, b)
        if issue_gather:
            if wait_scat:
                _wait_scatter(j - 1, 1 - b)
            _gather(j + 1, 1 - b)

    _gather(0, 0)
    _step(0, 0, wait_scat=False)
    _step(1, 1)
    _step(2, 0)

    @pl.loop(0, (ANCH - 5) // 2)
    def _(blk):
        j0 = 3 + blk * 2
        _step(j0, 1)
        _step(j0 + 1, 0)

    for j in range(3 + 2 * ((ANCH - 5) // 2), ANCH):
        _step(j, j % 2, issue_gather=(j < ANCH - 1))
    _wait_scatter(ANCH - 2, (ANCH - 2) % 2)
    _wait_scatter(ANCH - 1, (ANCH - 1) % 2)

    plsc.subcore_barrier()

    @pl.loop(0, DRAIN_C)
    def _(k):
        rr = r0 + k * ACH
        pltpu.sync_copy(acc_sh.at[pl.ds(rr, ACH)], rows[0])
        pltpu.sync_copy(rows[0], out_hbm.at[cid, pl.ds(rr, ACH)])


_RB = 1000
_GRID = N_NODES // _RB

_HIGH = jax.lax.Precision.HIGHEST
_DN = (((1,), (0,)), ((), ()))


def _row_spec(width):
    return pl.BlockSpec((_RB, width), lambda i: (i, 0))


def _full_spec(shape):
    return pl.BlockSpec(shape, lambda i: tuple(0 for _ in shape))


def _prescale_body(x_ref, dinv_ref, xs_ref):
    xs_ref[...] = x_ref[...] * dinv_ref[...]


_prescale = pl.pallas_call(
    _prescale_body,
    grid=(_GRID,),
    in_specs=[_row_spec(D), _row_spec(1)],
    out_specs=_row_spec(D),
    out_shape=jax.ShapeDtypeStruct((N_NODES, D), jnp.float32),
)


def _mid_body(p0_ref, p1_ref, x_ref, dinv_ref, dinv2_ref,
              w1_ref, b1_ref, w2_ref, y_ref, ys_ref):
    ax = (dinv_ref[...] * (p0_ref[...] + p1_ref[...])
          + dinv2_ref[...] * x_ref[...])
    h1 = lax.dot_general(ax, w1_ref[...], _DN, precision=_HIGH,
                         preferred_element_type=jnp.float32)
    h1 = jnp.maximum(h1 + b1_ref[...], 0.0)
    y = lax.dot_general(h1, w2_ref[...], _DN, precision=_HIGH,
                        preferred_element_type=jnp.float32)
    y_ref[...] = y
    ys_ref[...] = y * dinv_ref[...]


_mid = pl.pallas_call(
    _mid_body,
    grid=(_GRID,),
    in_specs=[_row_spec(D), _row_spec(D), _row_spec(D), _row_spec(1),
              _row_spec(1), _full_spec((D, 2 * D)), _full_spec((1, 2 * D)),
              _full_spec((2 * D, D))],
    out_specs=[_row_spec(D), _row_spec(D)],
    out_shape=[jax.ShapeDtypeStruct((N_NODES, D), jnp.float32),
               jax.ShapeDtypeStruct((N_NODES, D), jnp.float32)],
)


def _final_body(q0_ref, q1_ref, y_ref, dinv_ref, dinv2_ref, b2_ref, out_ref):
    agg = dinv_ref[...] * (q0_ref[...] + q1_ref[...])
    out_ref[...] = jnp.maximum(agg + dinv2_ref[...] * y_ref[...]
                               + b2_ref[...], 0.0)


_final = pl.pallas_call(
    _final_body,
    grid=(_GRID,),
    in_specs=[_row_spec(D), _row_spec(D), _row_spec(D), _row_spec(1),
              _row_spec(1), _full_spec((1, D))],
    out_specs=_row_spec(D),
    out_shape=jax.ShapeDtypeStruct((N_NODES, D), jnp.float32),
)


def kernel(x, edge_index, W1, b1, W2, b2):
    ei = edge_index.astype(jnp.int32)
    src3 = ei[0].reshape(NW, EPT)
    dst3 = ei[1].reshape(NW, EPT)
    deg_p = _deg_kernel(
        ei[1].reshape(NW * _NSLAB, _DSLAB, CHUNK)).reshape(NC, N_NODES)
    deg = deg_p[0] + deg_p[1] + 1.0
    dinv = lax.rsqrt(deg).reshape(N_NODES, 1)
    dinv2 = (1.0 / deg).reshape(N_NODES, 1)

    xs = _prescale(x, dinv)
    agg1 = _agg_kernel(xs, src3, dst3)
    y, ys = _mid(agg1[0], agg1[1], x, dinv, dinv2,
                 W1, b1.reshape(1, 2 * D), W2)
    agg2 = _agg_kernel(ys, src3, dst3)
    return _final(agg2[0], agg2[1], y, dinv, dinv2, b2.reshape(1, D))


# speedup vs baseline: 27.5689x; 1.0848x over previous
import functools

import jax
import jax.numpy as jnp
from jax import lax
from jax.experimental import pallas as pl
from jax.experimental.pallas import tpu as pltpu
from jax.experimental.pallas import tpu_sc as plsc

N_NODES = 10000
N_EDGES = 320000
D = 128
NC = 2
NS = 16
NW = NC * NS
EPT = N_EDGES // NW
CHUNK = 80
NCHUNK = EPT // CHUNK
ROWS_PT = 640
_LAST_R0 = N_NODES - ROWS_PT
ACH = 80
ANCH = EPT // ACH
DRAIN_C = ROWS_PT // ACH
_DSLAB = 25
_NSLAB = NCHUNK // _DSLAB

_MESH = plsc.VectorSubcoreMesh(
    core_axis_name="c", subcore_axis_name="s", num_cores=NC, num_subcores=NS
)


@functools.partial(
    pl.kernel,
    out_type=jax.ShapeDtypeStruct((NC * N_NODES,), jnp.float32),
    mesh=_MESH,
    scratch_types=[
        pltpu.VMEM((_DSLAB, CHUNK), jnp.int32),
        pltpu.VMEM((CHUNK,), jnp.float32),
        pltpu.VMEM((ROWS_PT,), jnp.float32),
        pltpu.VMEM_SHARED((N_NODES,), jnp.float32),
    ],
)
def _deg_kernel(dst_hbm, out_hbm, didx, ones_v, zbuf, acc_sh):
    cid = lax.axis_index("c")
    sid = lax.axis_index("s")
    wid = cid * NS + sid
    r0 = jnp.minimum(sid * ROWS_PT, _LAST_R0)

    @pl.loop(0, ROWS_PT, step=16)
    def _(i):
        zbuf[pl.ds(i, 16)] = jnp.zeros((16,), jnp.float32)

    @pl.loop(0, CHUNK, step=16)
    def _(i):
        ones_v[pl.ds(i, 16)] = jnp.full((16,), 1.0, jnp.float32)

    pltpu.sync_copy(zbuf, acc_sh.at[pl.ds(r0, ROWS_PT)])
    plsc.subcore_barrier()

    @pl.loop(0, _NSLAB)
    def _(s):
        pltpu.sync_copy(dst_hbm.at[wid * _NSLAB + s], didx)

        @pl.loop(0, _DSLAB)
        def _(j):
            pltpu.sync_copy(ones_v, acc_sh.at[didx.at[j]], add=True)

    plsc.subcore_barrier()
    pltpu.sync_copy(acc_sh.at[pl.ds(r0, ROWS_PT)], zbuf)
    pltpu.sync_copy(zbuf, out_hbm.at[pl.ds(cid * N_NODES + r0, ROWS_PT)])


@functools.partial(
    pl.kernel,
    out_type=jax.ShapeDtypeStruct((NC, N_NODES, D), jnp.float32),
    mesh=_MESH,
    scratch_types=[
        pltpu.VMEM((EPT,), jnp.int32),
        pltpu.VMEM((EPT,), jnp.int32),
        pltpu.VMEM((2, ACH, D), jnp.float32),
        pltpu.VMEM_SHARED((N_NODES, D), jnp.float32),
    ] + [pltpu.SemaphoreType.DMA] * 4,
)
def _agg_kernel(table_hbm, src_hbm, dst_hbm, out_hbm,
                sidx, didx, rowbuf, acc_sh, g0, g1, s0, s1):
    rows = (rowbuf.at[0], rowbuf.at[1])
    gsem = (g0, g1)
    ssem = (s0, s1)
    cid = lax.axis_index("c")
    sid = lax.axis_index("s")
    wid = cid * NS + sid
    r0 = jnp.minimum(sid * ROWS_PT, _LAST_R0)

    @pl.loop(0, ACH)
    def _(i):
        @pl.loop(0, D, step=16)
        def _(j):
            rowbuf[0, i, pl.ds(j, 16)] = jnp.zeros((16,), jnp.float32)

    @pl.loop(0, DRAIN_C)
    def _(k):
        pltpu.sync_copy(rows[0], acc_sh.at[pl.ds(r0 + k * ACH, ACH)])

    pltpu.sync_copy(src_hbm.at[wid], sidx)
    pltpu.sync_copy(dst_hbm.at[wid], didx)
    plsc.subcore_barrier()

    def _gather(j, b):
        pltpu.async_copy(table_hbm.at[sidx.at[pl.ds(j * ACH, ACH)]],
                         rows[b], gsem[b])

    def _wait_gather(j, b):
        pltpu.make_async_copy(table_hbm.at[sidx.at[pl.ds(j * ACH, ACH)]],
                              rows[b], gsem[b]).wait()

    def _scatter(j, b):
        pltpu.async_copy(rows[b], acc_sh.at[didx.at[pl.ds(j * ACH, ACH)]],
                         ssem[b], add=True)

    def _wait_scatter(j, b):
        pltpu.make_async_copy(rows[b], acc_sh.at[didx.at[pl.ds(j * ACH, ACH)]],
                              ssem[b]).wait()

    def _step(j, b, wait_scat=True, issue_gather=True):
        _wait_gather(j, b)
        _scatter(j, b)
        if issue_gather:
            if wait_scat:
                _wait_scatter(j - 1, 1 - b)
            _gather(j + 1, 1 - b)

    _gather(0, 0)
    _step(0, 0, wait_scat=False)
    _step(1, 1)
    _step(2, 0)

    @pl.loop(0, (ANCH - 5) // 2)
    def _(blk):
        j0 = 3 + blk * 2
        _step(j0, 1)
        _step(j0 + 1, 0)

    for j in range(3 + 2 * ((ANCH - 5) // 2), ANCH):
        _step(j, j % 2, issue_gather=(j < ANCH - 1))
    _wait_scatter(ANCH - 2, (ANCH - 2) % 2)
    _wait_scatter(ANCH - 1, (ANCH - 1) % 2)

    plsc.subcore_barrier()

    @pl.loop(0, DRAIN_C)
    def _(k):
        rr = r0 + k * ACH
        pltpu.sync_copy(acc_sh.at[pl.ds(rr, ACH)], rows[0])
        pltpu.sync_copy(rows[0], out_hbm.at[cid, pl.ds(rr, ACH)])


_RB = 1000
_GRID = N_NODES // _RB

_DN = (((1,), (0,)), ((), ()))


def _row_spec(width):
    return pl.BlockSpec((_RB, width), lambda i: (i, 0))


def _pair_spec(width):
    return pl.BlockSpec((NC, _RB, width), lambda i: (0, i, 0))


def _full_spec(shape):
    return pl.BlockSpec(shape, lambda i: tuple(0 for _ in shape))


def _prep_body(degp_ref, x_ref, xs_ref, dinv_ref, dinv2_ref):
    d = degp_ref[:, 0:1] + degp_ref[:, 1:2] + 1.0
    dc = lax.rsqrt(d)
    dinv_ref[...] = dc
    dinv2_ref[...] = 1.0 / d
    xs_ref[...] = x_ref[...] * dc


_prep = pl.pallas_call(
    _prep_body,
    grid=(_GRID,),
    in_specs=[pl.BlockSpec((_RB, NC), lambda i: (i, 0)), _row_spec(D)],
    out_specs=[_row_spec(D), _row_spec(1), _row_spec(1)],
    out_shape=[jax.ShapeDtypeStruct((N_NODES, D), jnp.float32),
               jax.ShapeDtypeStruct((N_NODES, 1), jnp.float32),
               jax.ShapeDtypeStruct((N_NODES, 1), jnp.float32)],
)


def _mid_body(pp_ref, x_ref, dinv_ref, dinv2_ref,
              w1_ref, b1_ref, w2_ref, y_ref, ys_ref):
    ax = (dinv_ref[...] * (pp_ref[0] + pp_ref[1])
          + dinv2_ref[...] * x_ref[...])
    h1 = lax.dot_general(ax, w1_ref[...], _DN,
                         preferred_element_type=jnp.float32)
    h1 = jnp.maximum(h1 + b1_ref[...], 0.0)
    y = lax.dot_general(h1, w2_ref[...], _DN,
                        preferred_element_type=jnp.float32)
    y_ref[...] = y
    ys_ref[...] = y * dinv_ref[...]


_mid = pl.pallas_call(
    _mid_body,
    grid=(_GRID,),
    in_specs=[_pair_spec(D), _row_spec(D), _row_spec(1),
              _row_spec(1), _full_spec((D, 2 * D)), _full_spec((1, 2 * D)),
              _full_spec((2 * D, D))],
    out_specs=[_row_spec(D), _row_spec(D)],
    out_shape=[jax.ShapeDtypeStruct((N_NODES, D), jnp.float32),
               jax.ShapeDtypeStruct((N_NODES, D), jnp.float32)],
)


def _final_body(qq_ref, y_ref, dinv_ref, dinv2_ref, b2_ref, out_ref):
    agg = dinv_ref[...] * (qq_ref[0] + qq_ref[1])
    out_ref[...] = jnp.maximum(agg + dinv2_ref[...] * y_ref[...]
                               + b2_ref[...], 0.0)


_final = pl.pallas_call(
    _final_body,
    grid=(_GRID,),
    in_specs=[_pair_spec(D), _row_spec(D), _row_spec(1),
              _row_spec(1), _full_spec((1, D))],
    out_specs=_row_spec(D),
    out_shape=jax.ShapeDtypeStruct((N_NODES, D), jnp.float32),
)


def kernel(x, edge_index, W1, b1, W2, b2):
    ei = edge_index.astype(jnp.int32)
    src3 = ei[0].reshape(NW, EPT)
    dst3 = ei[1].reshape(NW, EPT)
    deg_p = _deg_kernel(
        ei[1].reshape(NW * _NSLAB, _DSLAB, CHUNK)).reshape(NC, N_NODES)
    xs, dinv, dinv2 = _prep(deg_p.T, x)
    agg1 = _agg_kernel(xs, src3, dst3)
    y, ys = _mid(agg1, x, dinv, dinv2, W1, b1.reshape(1, 2 * D), W2)
    agg2 = _agg_kernel(ys, src3, dst3)
    return _final(agg2, y, dinv, dinv2, b2.reshape(1, D))


# speedup vs baseline: 28.4504x; 1.0320x over previous
import functools

import jax
import jax.numpy as jnp
from jax import lax
from jax.experimental import pallas as pl
from jax.experimental.pallas import tpu as pltpu
from jax.experimental.pallas import tpu_sc as plsc

N_NODES = 10000
N_EDGES = 320000
D = 128
NC = 2
NS = 16
NW = NC * NS
EPT = N_EDGES // NW
CHUNK = 80
NCHUNK = EPT // CHUNK
ROWS_PT = 640
_LAST_R0 = N_NODES - ROWS_PT
ACH = 80
ANCH = EPT // ACH
DRAIN_C = ROWS_PT // ACH
_DSLAB = 25
_NSLAB = NCHUNK // _DSLAB

_MESH = plsc.VectorSubcoreMesh(
    core_axis_name="c", subcore_axis_name="s", num_cores=NC, num_subcores=NS
)


@functools.partial(
    pl.kernel,
    out_type=jax.ShapeDtypeStruct((NC * N_NODES,), jnp.float32),
    mesh=_MESH,
    scratch_types=[
        pltpu.VMEM((EPT,), jnp.int32),
        pltpu.VMEM((CHUNK,), jnp.float32),
        pltpu.VMEM((ROWS_PT,), jnp.float32),
        pltpu.VMEM_SHARED((N_NODES,), jnp.float32),
    ],
)
def _deg_kernel(edges_hbm, out_hbm, didx, ones_v, zbuf, acc_sh):
    cid = lax.axis_index("c")
    sid = lax.axis_index("s")
    wid = cid * NS + sid
    r0 = jnp.minimum(sid * ROWS_PT, _LAST_R0)

    @pl.loop(0, ROWS_PT, step=16)
    def _(i):
        zbuf[pl.ds(i, 16)] = jnp.zeros((16,), jnp.float32)

    @pl.loop(0, CHUNK, step=16)
    def _(i):
        ones_v[pl.ds(i, 16)] = jnp.full((16,), 1.0, jnp.float32)

    pltpu.sync_copy(zbuf, acc_sh.at[pl.ds(r0, ROWS_PT)])
    pltpu.sync_copy(edges_hbm.at[NW + wid], didx)
    plsc.subcore_barrier()

    @pl.loop(0, NCHUNK)
    def _(j):
        pltpu.sync_copy(ones_v, acc_sh.at[didx.at[pl.ds(j * CHUNK, CHUNK)]],
                        add=True)

    plsc.subcore_barrier()
    pltpu.sync_copy(acc_sh.at[pl.ds(r0, ROWS_PT)], zbuf)
    pltpu.sync_copy(zbuf, out_hbm.at[pl.ds(cid * N_NODES + r0, ROWS_PT)])


@functools.partial(
    pl.kernel,
    out_type=jax.ShapeDtypeStruct((NC, N_NODES, D), jnp.float32),
    mesh=_MESH,
    scratch_types=[
        pltpu.VMEM((EPT,), jnp.int32),
        pltpu.VMEM((EPT,), jnp.int32),
        pltpu.VMEM((2, ACH, D), jnp.float32),
        pltpu.VMEM_SHARED((N_NODES, D), jnp.float32),
    ] + [pltpu.SemaphoreType.DMA] * 4,
)
def _agg_kernel(table_hbm, edges_hbm, out_hbm,
                sidx, didx, rowbuf, acc_sh, g0, g1, s0, s1):
    rows = (rowbuf.at[0], rowbuf.at[1])
    gsem = (g0, g1)
    ssem = (s0, s1)
    cid = lax.axis_index("c")
    sid = lax.axis_index("s")
    wid = cid * NS + sid
    r0 = jnp.minimum(sid * ROWS_PT, _LAST_R0)

    @pl.loop(0, ACH)
    def _(i):
        @pl.loop(0, D, step=16)
        def _(j):
            rowbuf[0, i, pl.ds(j, 16)] = jnp.zeros((16,), jnp.float32)

    @pl.loop(0, DRAIN_C)
    def _(k):
        pltpu.sync_copy(rows[0], acc_sh.at[pl.ds(r0 + k * ACH, ACH)])

    pltpu.sync_copy(edges_hbm.at[wid], sidx)
    pltpu.sync_copy(edges_hbm.at[NW + wid], didx)
    plsc.subcore_barrier()

    def _gather(j, b):
        pltpu.async_copy(table_hbm.at[sidx.at[pl.ds(j * ACH, ACH)]],
                         rows[b], gsem[b])

    def _wait_gather(j, b):
        pltpu.make_async_copy(table_hbm.at[sidx.at[pl.ds(j * ACH, ACH)]],
                              rows[b], gsem[b]).wait()

    def _scatter(j, b):
        pltpu.async_copy(rows[b], acc_sh.at[didx.at[pl.ds(j * ACH, ACH)]],
                         ssem[b], add=True)

    def _wait_scatter(j, b):
        pltpu.make_async_copy(rows[b], acc_sh.at[didx.at[pl.ds(j * ACH, ACH)]],
                              ssem[b]).wait()

    def _step(j, b, wait_scat=True, issue_gather=True):
        _wait_gather(j, b)
        _scatter(j, b)
        if issue_gather:
            if wait_scat:
                _wait_scatter(j - 1, 1 - b)
            _gather(j + 1, 1 - b)

    _gather(0, 0)
    _step(0, 0, wait_scat=False)
    _step(1, 1)
    _step(2, 0)

    @pl.loop(0, (ANCH - 5) // 2)
    def _(blk):
        j0 = 3 + blk * 2
        _step(j0, 1)
        _step(j0 + 1, 0)

    for j in range(3 + 2 * ((ANCH - 5) // 2), ANCH):
        _step(j, j % 2, issue_gather=(j < ANCH - 1))
    _wait_scatter(ANCH - 2, (ANCH - 2) % 2)
    _wait_scatter(ANCH - 1, (ANCH - 1) % 2)

    plsc.subcore_barrier()

    @pl.loop(0, DRAIN_C)
    def _(k):
        rr = r0 + k * ACH
        pltpu.sync_copy(acc_sh.at[pl.ds(rr, ACH)], rows[0])
        pltpu.sync_copy(rows[0], out_hbm.at[cid, pl.ds(rr, ACH)])


_RB = 1000
_GRID = N_NODES // _RB

_DN = (((1,), (0,)), ((), ()))


def _row_spec(width):
    return pl.BlockSpec((_RB, width), lambda i: (i, 0))


def _pair_spec(width):
    return pl.BlockSpec((NC, _RB, width), lambda i: (0, i, 0))


def _full_spec(shape):
    return pl.BlockSpec(shape, lambda i: tuple(0 for _ in shape))


def _prep_body(degp_ref, x_ref, xs_ref, dinv_ref, dinv2_ref):
    d = degp_ref[:, 0:1] + degp_ref[:, 1:2] + 1.0
    dc = lax.rsqrt(d)
    dinv_ref[...] = dc
    dinv2_ref[...] = 1.0 / d
    xs_ref[...] = x_ref[...] * dc


_prep = pl.pallas_call(
    _prep_body,
    grid=(_GRID,),
    in_specs=[pl.BlockSpec((_RB, NC), lambda i: (i, 0)), _row_spec(D)],
    out_specs=[_row_spec(D), _row_spec(1), _row_spec(1)],
    out_shape=[jax.ShapeDtypeStruct((N_NODES, D), jnp.float32),
               jax.ShapeDtypeStruct((N_NODES, 1), jnp.float32),
               jax.ShapeDtypeStruct((N_NODES, 1), jnp.float32)],
)


def _mid_body(pp_ref, x_ref, dinv_ref, dinv2_ref,
              w1_ref, b1_ref, w2_ref, y_ref, ys_ref):
    ax = (dinv_ref[...] * (pp_ref[0] + pp_ref[1])
          + dinv2_ref[...] * x_ref[...])
    h1 = lax.dot_general(ax, w1_ref[...], _DN,
                         preferred_element_type=jnp.float32)
    h1 = jnp.maximum(h1 + b1_ref[...], 0.0)
    y = lax.dot_general(h1, w2_ref[...], _DN,
                        preferred_element_type=jnp.float32)
    y_ref[...] = y
    ys_ref[...] = y * dinv_ref[...]


_mid = pl.pallas_call(
    _mid_body,
    grid=(_GRID,),
    in_specs=[_pair_spec(D), _row_spec(D), _row_spec(1),
              _row_spec(1), _full_spec((D, 2 * D)), _full_spec((1, 2 * D)),
              _full_spec((2 * D, D))],
    out_specs=[_row_spec(D), _row_spec(D)],
    out_shape=[jax.ShapeDtypeStruct((N_NODES, D), jnp.float32),
               jax.ShapeDtypeStruct((N_NODES, D), jnp.float32)],
)


def _final_body(qq_ref, y_ref, dinv_ref, dinv2_ref, b2_ref, out_ref):
    agg = dinv_ref[...] * (qq_ref[0] + qq_ref[1])
    out_ref[...] = jnp.maximum(agg + dinv2_ref[...] * y_ref[...]
                               + b2_ref[...], 0.0)


_final = pl.pallas_call(
    _final_body,
    grid=(_GRID,),
    in_specs=[_pair_spec(D), _row_spec(D), _row_spec(1),
              _row_spec(1), _full_spec((1, D))],
    out_specs=_row_spec(D),
    out_shape=jax.ShapeDtypeStruct((N_NODES, D), jnp.float32),
)


def kernel(x, edge_index, W1, b1, W2, b2):
    edges = edge_index.astype(jnp.int32).reshape(2 * NW, EPT)
    deg_p = _deg_kernel(edges).reshape(NC, N_NODES)
    xs, dinv, dinv2 = _prep(deg_p.T, x)
    agg1 = _agg_kernel(xs, edges)
    y, ys = _mid(agg1, x, dinv, dinv2, W1, b1.reshape(1, 2 * D), W2)
    agg2 = _agg_kernel(ys, edges)
    return _final(agg2, y, dinv, dinv2, b2.reshape(1, D))


# speedup vs baseline: 28.8711x; 1.0148x over previous
import functools

import jax
import jax.numpy as jnp
from jax import lax
from jax.experimental import pallas as pl
from jax.experimental.pallas import tpu as pltpu
from jax.experimental.pallas import tpu_sc as plsc

N_NODES = 10000
N_EDGES = 320000
D = 128
NC = 2
NS = 16
NW = NC * NS
EPT = N_EDGES // NW
CHUNK = 80
NCHUNK = EPT // CHUNK
ROWS_PT = 640
_LAST_R0 = N_NODES - ROWS_PT
ACH = 80
ANCH = EPT // ACH
DRAIN_C = ROWS_PT // ACH
_DSLAB = 25
_NSLAB = NCHUNK // _DSLAB

_MESH = plsc.VectorSubcoreMesh(
    core_axis_name="c", subcore_axis_name="s", num_cores=NC, num_subcores=NS
)


@functools.partial(
    pl.kernel,
    out_type=jax.ShapeDtypeStruct((NC * N_NODES,), jnp.float32),
    mesh=_MESH,
    scratch_types=[
        pltpu.VMEM((EPT,), jnp.int32),
        pltpu.VMEM((CHUNK,), jnp.float32),
        pltpu.VMEM((ROWS_PT,), jnp.float32),
        pltpu.VMEM_SHARED((N_NODES,), jnp.float32),
    ],
)
def _deg_kernel(edges_hbm, out_hbm, didx, ones_v, zbuf, acc_sh):
    cid = lax.axis_index("c")
    sid = lax.axis_index("s")
    wid = cid * NS + sid
    r0 = jnp.minimum(sid * ROWS_PT, _LAST_R0)

    @pl.loop(0, ROWS_PT, step=16)
    def _(i):
        zbuf[pl.ds(i, 16)] = jnp.zeros((16,), jnp.float32)

    @pl.loop(0, CHUNK, step=16)
    def _(i):
        ones_v[pl.ds(i, 16)] = jnp.full((16,), 1.0, jnp.float32)

    pltpu.sync_copy(zbuf, acc_sh.at[pl.ds(r0, ROWS_PT)])
    pltpu.sync_copy(edges_hbm.at[NW + wid], didx)
    plsc.subcore_barrier()

    @pl.loop(0, NCHUNK)
    def _(j):
        pltpu.sync_copy(ones_v, acc_sh.at[didx.at[pl.ds(j * CHUNK, CHUNK)]],
                        add=True)

    plsc.subcore_barrier()
    pltpu.sync_copy(acc_sh.at[pl.ds(r0, ROWS_PT)], zbuf)
    pltpu.sync_copy(zbuf, out_hbm.at[pl.ds(cid * N_NODES + r0, ROWS_PT)])


@functools.partial(
    pl.kernel,
    out_type=jax.ShapeDtypeStruct((NC, N_NODES, D), jnp.float32),
    mesh=_MESH,
    scratch_types=[
        pltpu.VMEM((EPT,), jnp.int32),
        pltpu.VMEM((EPT,), jnp.int32),
        pltpu.VMEM((2, ACH, D), jnp.float32),
        pltpu.VMEM_SHARED((N_NODES, D), jnp.float32),
    ] + [pltpu.SemaphoreType.DMA] * 4,
)
def _agg_kernel(table_hbm, edges_hbm, out_hbm,
                sidx, didx, rowbuf, acc_sh, g0, g1, s0, s1):
    rows = (rowbuf.at[0], rowbuf.at[1])
    gsem = (g0, g1)
    ssem = (s0, s1)
    cid = lax.axis_index("c")
    sid = lax.axis_index("s")
    wid = cid * NS + sid
    r0 = jnp.minimum(sid * ROWS_PT, _LAST_R0)

    @pl.loop(0, ACH)
    def _(i):
        @pl.loop(0, D, step=16)
        def _(j):
            rowbuf[0, i, pl.ds(j, 16)] = jnp.zeros((16,), jnp.float32)

    @pl.loop(0, DRAIN_C)
    def _(k):
        pltpu.sync_copy(rows[0], acc_sh.at[pl.ds(r0 + k * ACH, ACH)])

    pltpu.sync_copy(edges_hbm.at[wid], sidx)
    pltpu.sync_copy(edges_hbm.at[NW + wid], didx)
    plsc.subcore_barrier()

    def _gather(j, b):
        pltpu.async_copy(table_hbm.at[sidx.at[pl.ds(j * ACH, ACH)]],
                         rows[b], gsem[b])

    def _wait_gather(j, b):
        pltpu.make_async_copy(table_hbm.at[sidx.at[pl.ds(j * ACH, ACH)]],
                              rows[b], gsem[b]).wait()

    def _scatter(j, b):
        pltpu.async_copy(rows[b], acc_sh.at[didx.at[pl.ds(j * ACH, ACH)]],
                         ssem[b], add=True)

    def _wait_scatter(j, b):
        pltpu.make_async_copy(rows[b], acc_sh.at[didx.at[pl.ds(j * ACH, ACH)]],
                              ssem[b]).wait()

    def _step(j, b, wait_scat=True, issue_gather=True):
        _wait_gather(j, b)
        _scatter(j, b)
        if issue_gather:
            if wait_scat:
                _wait_scatter(j - 1, 1 - b)
            _gather(j + 1, 1 - b)

    _gather(0, 0)
    _step(0, 0, wait_scat=False)
    _step(1, 1)
    _step(2, 0)

    @pl.loop(0, (ANCH - 5) // 2)
    def _(blk):
        j0 = 3 + blk * 2
        _step(j0, 1)
        _step(j0 + 1, 0)

    for j in range(3 + 2 * ((ANCH - 5) // 2), ANCH):
        _step(j, j % 2, issue_gather=(j < ANCH - 1))
    _wait_scatter(ANCH - 2, (ANCH - 2) % 2)
    _wait_scatter(ANCH - 1, (ANCH - 1) % 2)

    plsc.subcore_barrier()

    @pl.loop(0, DRAIN_C)
    def _(k):
        rr = r0 + k * ACH
        pltpu.sync_copy(acc_sh.at[pl.ds(rr, ACH)], rows[0])
        pltpu.sync_copy(rows[0], out_hbm.at[cid, pl.ds(rr, ACH)])


_RB = 2000
_GRID = N_NODES // _RB

_DN = (((1,), (0,)), ((), ()))


def _row_spec(width):
    return pl.BlockSpec((_RB, width), lambda i: (i, 0))


def _pair_spec(width):
    return pl.BlockSpec((NC, _RB, width), lambda i: (0, i, 0))


def _full_spec(shape):
    return pl.BlockSpec(shape, lambda i: tuple(0 for _ in shape))


def _prep_body(degp_ref, x_ref, xs_ref, dinv_ref, dinv2_ref):
    d = degp_ref[:, 0:1] + degp_ref[:, 1:2] + 1.0
    dc = lax.rsqrt(d)
    dinv_ref[...] = dc
    dinv2_ref[...] = 1.0 / d
    xs_ref[...] = x_ref[...] * dc


_prep = pl.pallas_call(
    _prep_body,
    grid=(_GRID,),
    in_specs=[pl.BlockSpec((_RB, NC), lambda i: (i, 0)), _row_spec(D)],
    out_specs=[_row_spec(D), _row_spec(1), _row_spec(1)],
    out_shape=[jax.ShapeDtypeStruct((N_NODES, D), jnp.float32),
               jax.ShapeDtypeStruct((N_NODES, 1), jnp.float32),
               jax.ShapeDtypeStruct((N_NODES, 1), jnp.float32)],
)


def _mid_body(pp_ref, x_ref, dinv_ref, dinv2_ref,
              w1_ref, b1_ref, w2_ref, y_ref, ys_ref):
    ax = (dinv_ref[...] * (pp_ref[0] + pp_ref[1])
          + dinv2_ref[...] * x_ref[...])
    h1 = lax.dot_general(ax, w1_ref[...], _DN,
                         preferred_element_type=jnp.float32)
    h1 = jnp.maximum(h1 + b1_ref[...], 0.0)
    y = lax.dot_general(h1, w2_ref[...], _DN,
                        preferred_element_type=jnp.float32)
    y_ref[...] = y
    ys_ref[...] = y * dinv_ref[...]


_mid = pl.pallas_call(
    _mid_body,
    grid=(_GRID,),
    in_specs=[_pair_spec(D), _row_spec(D), _row_spec(1),
              _row_spec(1), _full_spec((D, 2 * D)), _full_spec((1, 2 * D)),
              _full_spec((2 * D, D))],
    out_specs=[_row_spec(D), _row_spec(D)],
    out_shape=[jax.ShapeDtypeStruct((N_NODES, D), jnp.float32),
               jax.ShapeDtypeStruct((N_NODES, D), jnp.float32)],
)


def _final_body(qq_ref, y_ref, dinv_ref, dinv2_ref, b2_ref, out_ref):
    agg = dinv_ref[...] * (qq_ref[0] + qq_ref[1])
    out_ref[...] = jnp.maximum(agg + dinv2_ref[...] * y_ref[...]
                               + b2_ref[...], 0.0)


_final = pl.pallas_call(
    _final_body,
    grid=(_GRID,),
    in_specs=[_pair_spec(D), _row_spec(D), _row_spec(1),
              _row_spec(1), _full_spec((1, D))],
    out_specs=_row_spec(D),
    out_shape=jax.ShapeDtypeStruct((N_NODES, D), jnp.float32),
)


def kernel(x, edge_index, W1, b1, W2, b2):
    edges = edge_index.astype(jnp.int32).reshape(2 * NW, EPT)
    deg_p = _deg_kernel(edges).reshape(NC, N_NODES)
    xs, dinv, dinv2 = _prep(deg_p.T, x)
    agg1 = _agg_kernel(xs, edges)
    y, ys = _mid(agg1, x, dinv, dinv2, W1, b1.reshape(1, 2 * D), W2)
    agg2 = _agg_kernel(ys, edges)
    return _final(agg2, y, dinv, dinv2, b2.reshape(1, D))


# speedup vs baseline: 29.2350x; 1.0126x over previous
import functools

import jax
import jax.numpy as jnp
from jax import lax
from jax.experimental import pallas as pl
from jax.experimental.pallas import tpu as pltpu
from jax.experimental.pallas import tpu_sc as plsc

N_NODES = 10000
N_EDGES = 320000
D = 128
NC = 2
NS = 16
NW = NC * NS
EPT = N_EDGES // NW
CHUNK = 80
NCHUNK = EPT // CHUNK
ROWS_PT = 640
_LAST_R0 = N_NODES - ROWS_PT
ACH = 80
ANCH = EPT // ACH
DRAIN_C = ROWS_PT // ACH
_DSLAB = 25
_NSLAB = NCHUNK // _DSLAB

_MESH = plsc.VectorSubcoreMesh(
    core_axis_name="c", subcore_axis_name="s", num_cores=NC, num_subcores=NS
)


@functools.partial(
    pl.kernel,
    out_type=jax.ShapeDtypeStruct((NC * N_NODES,), jnp.float32),
    mesh=_MESH,
    scratch_types=[
        pltpu.VMEM((EPT,), jnp.int32),
        pltpu.VMEM((CHUNK,), jnp.float32),
        pltpu.VMEM((ROWS_PT,), jnp.float32),
        pltpu.VMEM_SHARED((N_NODES,), jnp.float32),
        pltpu.SemaphoreType.DMA,
    ],
)
def _deg_kernel(edges_hbm, out_hbm, didx, ones_v, zbuf, acc_sh, isem):
    cid = lax.axis_index("c")
    sid = lax.axis_index("s")
    wid = cid * NS + sid
    r0 = jnp.minimum(sid * ROWS_PT, _LAST_R0)

    pltpu.async_copy(edges_hbm.at[NW + wid], didx, isem)

    @pl.loop(0, ROWS_PT, step=16)
    def _(i):
        zbuf[pl.ds(i, 16)] = jnp.zeros((16,), jnp.float32)

    @pl.loop(0, CHUNK, step=16)
    def _(i):
        ones_v[pl.ds(i, 16)] = jnp.full((16,), 1.0, jnp.float32)

    pltpu.sync_copy(zbuf, acc_sh.at[pl.ds(r0, ROWS_PT)])
    pltpu.make_async_copy(edges_hbm.at[NW + wid], didx, isem).wait()
    plsc.subcore_barrier()

    @pl.loop(0, NCHUNK)
    def _(j):
        pltpu.sync_copy(ones_v, acc_sh.at[didx.at[pl.ds(j * CHUNK, CHUNK)]],
                        add=True)

    plsc.subcore_barrier()
    pltpu.sync_copy(acc_sh.at[pl.ds(r0, ROWS_PT)], zbuf)
    pltpu.sync_copy(zbuf, out_hbm.at[pl.ds(cid * N_NODES + r0, ROWS_PT)])


@functools.partial(
    pl.kernel,
    out_type=jax.ShapeDtypeStruct((NC, N_NODES, D), jnp.float32),
    mesh=_MESH,
    scratch_types=[
        pltpu.VMEM((EPT,), jnp.int32),
        pltpu.VMEM((EPT,), jnp.int32),
        pltpu.VMEM((2, ACH, D), jnp.float32),
        pltpu.VMEM_SHARED((N_NODES, D), jnp.float32),
    ] + [pltpu.SemaphoreType.DMA] * 4,
)
def _agg_kernel(table_hbm, edges_hbm, out_hbm,
                sidx, didx, rowbuf, acc_sh, g0, g1, s0, s1):
    rows = (rowbuf.at[0], rowbuf.at[1])
    gsem = (g0, g1)
    ssem = (s0, s1)
    cid = lax.axis_index("c")
    sid = lax.axis_index("s")
    wid = cid * NS + sid
    r0 = jnp.minimum(sid * ROWS_PT, _LAST_R0)

    pltpu.async_copy(edges_hbm.at[wid], sidx, g0)
    pltpu.async_copy(edges_hbm.at[NW + wid], didx, g1)

    @pl.loop(0, ACH)
    def _(i):
        @pl.loop(0, D, step=16)
        def _(j):
            rowbuf[0, i, pl.ds(j, 16)] = jnp.zeros((16,), jnp.float32)

    @pl.loop(0, DRAIN_C)
    def _(k):
        pltpu.sync_copy(rows[0], acc_sh.at[pl.ds(r0 + k * ACH, ACH)])

    pltpu.make_async_copy(edges_hbm.at[wid], sidx, g0).wait()
    pltpu.make_async_copy(edges_hbm.at[NW + wid], didx, g1).wait()
    plsc.subcore_barrier()

    def _gather(j, b):
        pltpu.async_copy(table_hbm.at[sidx.at[pl.ds(j * ACH, ACH)]],
                         rows[b], gsem[b])

    def _wait_gather(j, b):
        pltpu.make_async_copy(table_hbm.at[sidx.at[pl.ds(j * ACH, ACH)]],
                              rows[b], gsem[b]).wait()

    def _scatter(j, b):
        pltpu.async_copy(rows[b], acc_sh.at[didx.at[pl.ds(j * ACH, ACH)]],
                         ssem[b], add=True)

    def _wait_scatter(j, b):
        pltpu.make_async_copy(rows[b], acc_sh.at[didx.at[pl.ds(j * ACH, ACH)]],
                              ssem[b]).wait()

    def _step(j, b, wait_scat=True, issue_gather=True):
        _wait_gather(j, b)
        _scatter(j, b)
        if issue_gather:
            if wait_scat:
                _wait_scatter(j - 1, 1 - b)
            _gather(j + 1, 1 - b)

    _gather(0, 0)
    _step(0, 0, wait_scat=False)
    _step(1, 1)
    _step(2, 0)

    @pl.loop(0, (ANCH - 5) // 2)
    def _(blk):
        j0 = 3 + blk * 2
        _step(j0, 1)
        _step(j0 + 1, 0)

    for j in range(3 + 2 * ((ANCH - 5) // 2), ANCH):
        _step(j, j % 2, issue_gather=(j < ANCH - 1))
    _wait_scatter(ANCH - 2, (ANCH - 2) % 2)
    _wait_scatter(ANCH - 1, (ANCH - 1) % 2)

    plsc.subcore_barrier()

    @pl.loop(0, DRAIN_C)
    def _(k):
        rr = r0 + k * ACH
        pltpu.sync_copy(acc_sh.at[pl.ds(rr, ACH)], rows[0])
        pltpu.sync_copy(rows[0], out_hbm.at[cid, pl.ds(rr, ACH)])


_RB = 2000
_GRID = N_NODES // _RB

_DN = (((1,), (0,)), ((), ()))


def _row_spec(width):
    return pl.BlockSpec((_RB, width), lambda i: (i, 0))


def _pair_spec(width):
    return pl.BlockSpec((NC, _RB, width), lambda i: (0, i, 0))


def _full_spec(shape):
    return pl.BlockSpec(shape, lambda i: tuple(0 for _ in shape))


def _prep_body(degp_ref, x_ref, xs_ref, dinv_ref, dinv2_ref):
    d = degp_ref[:, 0:1] + degp_ref[:, 1:2] + 1.0
    dc = lax.rsqrt(d)
    dinv_ref[...] = dc
    dinv2_ref[...] = 1.0 / d
    xs_ref[...] = x_ref[...] * dc


_prep = pl.pallas_call(
    _prep_body,
    grid=(_GRID,),
    in_specs=[pl.BlockSpec((_RB, NC), lambda i: (i, 0)), _row_spec(D)],
    out_specs=[_row_spec(D), _row_spec(1), _row_spec(1)],
    out_shape=[jax.ShapeDtypeStruct((N_NODES, D), jnp.float32),
               jax.ShapeDtypeStruct((N_NODES, 1), jnp.float32),
               jax.ShapeDtypeStruct((N_NODES, 1), jnp.float32)],
)


def _mid_body(pp_ref, x_ref, dinv_ref, dinv2_ref,
              w1_ref, b1_ref, w2_ref, y_ref, ys_ref):
    ax = (dinv_ref[...] * (pp_ref[0] + pp_ref[1])
          + dinv2_ref[...] * x_ref[...])
    h1 = lax.dot_general(ax, w1_ref[...], _DN,
                         preferred_element_type=jnp.float32)
    h1 = jnp.maximum(h1 + b1_ref[...], 0.0)
    y = lax.dot_general(h1, w2_ref[...], _DN,
                        preferred_element_type=jnp.float32)
    y_ref[...] = y
    ys_ref[...] = y * dinv_ref[...]


_mid = pl.pallas_call(
    _mid_body,
    grid=(_GRID,),
    in_specs=[_pair_spec(D), _row_spec(D), _row_spec(1),
              _row_spec(1), _full_spec((D, 2 * D)), _full_spec((1, 2 * D)),
              _full_spec((2 * D, D))],
    out_specs=[_row_spec(D), _row_spec(D)],
    out_shape=[jax.ShapeDtypeStruct((N_NODES, D), jnp.float32),
               jax.ShapeDtypeStruct((N_NODES, D), jnp.float32)],
)


def _final_body(qq_ref, y_ref, dinv_ref, dinv2_ref, b2_ref, out_ref):
    agg = dinv_ref[...] * (qq_ref[0] + qq_ref[1])
    out_ref[...] = jnp.maximum(agg + dinv2_ref[...] * y_ref[...]
                               + b2_ref[...], 0.0)


_final = pl.pallas_call(
    _final_body,
    grid=(_GRID,),
    in_specs=[_pair_spec(D), _row_spec(D), _row_spec(1),
              _row_spec(1), _full_spec((1, D))],
    out_specs=_row_spec(D),
    out_shape=jax.ShapeDtypeStruct((N_NODES, D), jnp.float32),
)


def kernel(x, edge_index, W1, b1, W2, b2):
    edges = edge_index.astype(jnp.int32).reshape(2 * NW, EPT)
    deg_p = _deg_kernel(edges).reshape(NC, N_NODES)
    xs, dinv, dinv2 = _prep(deg_p.T, x)
    agg1 = _agg_kernel(xs, edges)
    y, ys = _mid(agg1, x, dinv, dinv2, W1, b1.reshape(1, 2 * D), W2)
    agg2 = _agg_kernel(ys, edges)
    return _final(agg2, y, dinv, dinv2, b2.reshape(1, D))


# speedup vs baseline: 29.5625x; 1.0112x over previous
import functools

import jax
import jax.numpy as jnp
from jax import lax
from jax.experimental import pallas as pl
from jax.experimental.pallas import tpu as pltpu
from jax.experimental.pallas import tpu_sc as plsc

N_NODES = 10000
N_EDGES = 320000
D = 128
NC = 2
NS = 16
NW = NC * NS
EPT = N_EDGES // NW
CHUNK = 80
NCHUNK = EPT // CHUNK
ROWS_PT = 640
_LAST_R0 = N_NODES - ROWS_PT
ACH = 80
ANCH = EPT // ACH
DRAIN_C = ROWS_PT // ACH
_DSLAB = 25
_NSLAB = NCHUNK // _DSLAB

_MESH = plsc.VectorSubcoreMesh(
    core_axis_name="c", subcore_axis_name="s", num_cores=NC, num_subcores=NS
)


@functools.partial(
    pl.kernel,
    out_type=jax.ShapeDtypeStruct((NC * N_NODES,), jnp.float32),
    mesh=_MESH,
    scratch_types=[
        pltpu.VMEM((EPT,), jnp.int32),
        pltpu.VMEM((CHUNK,), jnp.float32),
        pltpu.VMEM((ROWS_PT,), jnp.float32),
        pltpu.VMEM_SHARED((N_NODES,), jnp.float32),
        pltpu.SemaphoreType.DMA,
    ],
)
def _deg_kernel(edges_hbm, out_hbm, didx, ones_v, zbuf, acc_sh, isem):
    cid = lax.axis_index("c")
    sid = lax.axis_index("s")
    wid = cid * NS + sid
    r0 = jnp.minimum(sid * ROWS_PT, _LAST_R0)

    pltpu.async_copy(edges_hbm.at[NW + wid], didx, isem)

    @pl.loop(0, ROWS_PT, step=16)
    def _(i):
        zbuf[pl.ds(i, 16)] = jnp.zeros((16,), jnp.float32)

    @pl.loop(0, CHUNK, step=16)
    def _(i):
        ones_v[pl.ds(i, 16)] = jnp.full((16,), 1.0, jnp.float32)

    pltpu.sync_copy(zbuf, acc_sh.at[pl.ds(r0, ROWS_PT)])
    pltpu.make_async_copy(edges_hbm.at[NW + wid], didx, isem).wait()
    plsc.subcore_barrier()

    @pl.loop(0, NCHUNK)
    def _(j):
        pltpu.sync_copy(ones_v, acc_sh.at[didx.at[pl.ds(j * CHUNK, CHUNK)]],
                        add=True)

    plsc.subcore_barrier()
    pltpu.sync_copy(acc_sh.at[pl.ds(r0, ROWS_PT)], zbuf)
    pltpu.sync_copy(zbuf, out_hbm.at[pl.ds(cid * N_NODES + r0, ROWS_PT)])


@functools.partial(
    pl.kernel,
    out_type=jax.ShapeDtypeStruct((NC, N_NODES, D), jnp.float32),
    mesh=_MESH,
    scratch_types=[
        pltpu.VMEM((EPT,), jnp.int32),
        pltpu.VMEM((EPT,), jnp.int32),
        pltpu.VMEM((2, ACH, D), jnp.float32),
        pltpu.VMEM_SHARED((N_NODES, D), jnp.float32),
    ] + [pltpu.SemaphoreType.DMA] * 4,
)
def _agg_kernel(table_hbm, edges_hbm, out_hbm,
                sidx, didx, rowbuf, acc_sh, g0, g1, s0, s1):
    rows = (rowbuf.at[0], rowbuf.at[1])
    gsem = (g0, g1)
    ssem = (s0, s1)
    cid = lax.axis_index("c")
    sid = lax.axis_index("s")
    wid = cid * NS + sid
    r0 = jnp.minimum(sid * ROWS_PT, _LAST_R0)

    pltpu.async_copy(edges_hbm.at[wid], sidx, g0)
    pltpu.async_copy(edges_hbm.at[NW + wid], didx, g1)

    @pl.loop(0, ACH)
    def _(i):
        @pl.loop(0, D, step=16)
        def _(j):
            rowbuf[1, i, pl.ds(j, 16)] = jnp.zeros((16,), jnp.float32)

    def _gather(j, b):
        pltpu.async_copy(table_hbm.at[sidx.at[pl.ds(j * ACH, ACH)]],
                         rows[b], gsem[b])

    def _wait_gather(j, b):
        pltpu.make_async_copy(table_hbm.at[sidx.at[pl.ds(j * ACH, ACH)]],
                              rows[b], gsem[b]).wait()

    def _scatter(j, b):
        pltpu.async_copy(rows[b], acc_sh.at[didx.at[pl.ds(j * ACH, ACH)]],
                         ssem[b], add=True)

    def _wait_scatter(j, b):
        pltpu.make_async_copy(rows[b], acc_sh.at[didx.at[pl.ds(j * ACH, ACH)]],
                              ssem[b]).wait()

    def _step(j, b, wait_scat=True, issue_gather=True):
        _wait_gather(j, b)
        _scatter(j, b)
        if issue_gather:
            if wait_scat:
                _wait_scatter(j - 1, 1 - b)
            _gather(j + 1, 1 - b)

    pltpu.make_async_copy(edges_hbm.at[wid], sidx, g0).wait()
    _gather(0, 0)

    @pl.loop(0, DRAIN_C)
    def _(k):
        pltpu.sync_copy(rows[1], acc_sh.at[pl.ds(r0 + k * ACH, ACH)])

    pltpu.make_async_copy(edges_hbm.at[NW + wid], didx, g1).wait()
    plsc.subcore_barrier()

    _step(0, 0, wait_scat=False)
    _step(1, 1)
    _step(2, 0)

    @pl.loop(0, (ANCH - 5) // 2)
    def _(blk):
        j0 = 3 + blk * 2
        _step(j0, 1)
        _step(j0 + 1, 0)

    for j in range(3 + 2 * ((ANCH - 5) // 2), ANCH):
        _step(j, j % 2, issue_gather=(j < ANCH - 1))
    _wait_scatter(ANCH - 2, (ANCH - 2) % 2)
    _wait_scatter(ANCH - 1, (ANCH - 1) % 2)

    plsc.subcore_barrier()

    def _dfill(k):
        pltpu.async_copy(acc_sh.at[pl.ds(r0 + k * ACH, ACH)], rows[k % 2],
                         gsem[k % 2])

    def _dpush(k):
        pltpu.async_copy(rows[k % 2], out_hbm.at[cid, pl.ds(r0 + k * ACH, ACH)],
                         ssem[k % 2])

    _dfill(0)
    for k in range(DRAIN_C):
        pltpu.make_async_copy(acc_sh.at[pl.ds(r0 + k * ACH, ACH)],
                              rows[k % 2], gsem[k % 2]).wait()
        _dpush(k)
        if k + 1 < DRAIN_C:
            if k >= 1:
                pltpu.make_async_copy(
                    rows[(k - 1) % 2],
                    out_hbm.at[cid, pl.ds(r0 + (k - 1) * ACH, ACH)],
                    ssem[(k - 1) % 2]).wait()
            _dfill(k + 1)
    for k in (DRAIN_C - 2, DRAIN_C - 1):
        pltpu.make_async_copy(rows[k % 2],
                              out_hbm.at[cid, pl.ds(r0 + k * ACH, ACH)],
                              ssem[k % 2]).wait()


_RB = 2000
_GRID = N_NODES // _RB

_DN = (((1,), (0,)), ((), ()))


def _row_spec(width):
    return pl.BlockSpec((_RB, width), lambda i: (i, 0))


def _pair_spec(width):
    return pl.BlockSpec((NC, _RB, width), lambda i: (0, i, 0))


def _full_spec(shape):
    return pl.BlockSpec(shape, lambda i: tuple(0 for _ in shape))


def _prep_body(degp_ref, x_ref, xs_ref, dinv_ref, dinv2_ref):
    d = degp_ref[:, 0:1] + degp_ref[:, 1:2] + 1.0
    dc = lax.rsqrt(d)
    dinv_ref[...] = dc
    dinv2_ref[...] = 1.0 / d
    xs_ref[...] = x_ref[...] * dc


_prep = pl.pallas_call(
    _prep_body,
    grid=(_GRID,),
    in_specs=[pl.BlockSpec((_RB, NC), lambda i: (i, 0)), _row_spec(D)],
    out_specs=[_row_spec(D), _row_spec(1), _row_spec(1)],
    out_shape=[jax.ShapeDtypeStruct((N_NODES, D), jnp.float32),
               jax.ShapeDtypeStruct((N_NODES, 1), jnp.float32),
               jax.ShapeDtypeStruct((N_NODES, 1), jnp.float32)],
)


def _mid_body(pp_ref, x_ref, dinv_ref, dinv2_ref,
              w1_ref, b1_ref, w2_ref, y_ref, ys_ref):
    ax = (dinv_ref[...] * (pp_ref[0] + pp_ref[1])
          + dinv2_ref[...] * x_ref[...])
    h1 = lax.dot_general(ax, w1_ref[...], _DN,
                         preferred_element_type=jnp.float32)
    h1 = jnp.maximum(h1 + b1_ref[...], 0.0)
    y = lax.dot_general(h1, w2_ref[...], _DN,
                        preferred_element_type=jnp.float32)
    y_ref[...] = y
    ys_ref[...] = y * dinv_ref[...]


_mid = pl.pallas_call(
    _mid_body,
    grid=(_GRID,),
    in_specs=[_pair_spec(D), _row_spec(D), _row_spec(1),
              _row_spec(1), _full_spec((D, 2 * D)), _full_spec((1, 2 * D)),
              _full_spec((2 * D, D))],
    out_specs=[_row_spec(D), _row_spec(D)],
    out_shape=[jax.ShapeDtypeStruct((N_NODES, D), jnp.float32),
               jax.ShapeDtypeStruct((N_NODES, D), jnp.float32)],
)


def _final_body(qq_ref, y_ref, dinv_ref, dinv2_ref, b2_ref, out_ref):
    agg = dinv_ref[...] * (qq_ref[0] + qq_ref[1])
    out_ref[...] = jnp.maximum(agg + dinv2_ref[...] * y_ref[...]
                               + b2_ref[...], 0.0)


_final = pl.pallas_call(
    _final_body,
    grid=(_GRID,),
    in_specs=[_pair_spec(D), _row_spec(D), _row_spec(1),
              _row_spec(1), _full_spec((1, D))],
    out_specs=_row_spec(D),
    out_shape=jax.ShapeDtypeStruct((N_NODES, D), jnp.float32),
)


def kernel(x, edge_index, W1, b1, W2, b2):
    edges = edge_index.astype(jnp.int32).reshape(2 * NW, EPT)
    deg_p = _deg_kernel(edges).reshape(NC, N_NODES)
    xs, dinv, dinv2 = _prep(deg_p.T, x)
    agg1 = _agg_kernel(xs, edges)
    y, ys = _mid(agg1, x, dinv, dinv2, W1, b1.reshape(1, 2 * D), W2)
    agg2 = _agg_kernel(ys, edges)
    return _final(agg2, y, dinv, dinv2, b2.reshape(1, D))


# speedup vs baseline: 30.0143x; 1.0153x over previous
import functools

import jax
import jax.numpy as jnp
from jax import lax
from jax.experimental import pallas as pl
from jax.experimental.pallas import tpu as pltpu
from jax.experimental.pallas import tpu_sc as plsc

N_NODES = 10000
N_EDGES = 320000
D = 128
NC = 2
NS = 16
NW = NC * NS
EPT = N_EDGES // NW
CHUNK = 80
NCHUNK = EPT // CHUNK
ROWS_PT = 640
_LAST_R0 = N_NODES - ROWS_PT
ACH = 80
ANCH = EPT // ACH
DRAIN_C = ROWS_PT // ACH
_DSLAB = 25
_NSLAB = NCHUNK // _DSLAB

_MESH = plsc.VectorSubcoreMesh(
    core_axis_name="c", subcore_axis_name="s", num_cores=NC, num_subcores=NS
)


@functools.partial(
    pl.kernel,
    out_type=jax.ShapeDtypeStruct((NC * N_NODES,), jnp.float32),
    mesh=_MESH,
    scratch_types=[
        pltpu.VMEM((EPT,), jnp.int32),
        pltpu.VMEM((CHUNK,), jnp.float32),
        pltpu.VMEM((ROWS_PT,), jnp.float32),
        pltpu.VMEM_SHARED((N_NODES,), jnp.float32),
        pltpu.SemaphoreType.DMA,
        pltpu.SemaphoreType.DMA,
    ],
)
def _deg_kernel(edges_hbm, out_hbm, didx, ones_v, zbuf, acc_sh, d0, d1):
    cid = lax.axis_index("c")
    sid = lax.axis_index("s")
    wid = cid * NS + sid
    r0 = jnp.minimum(sid * ROWS_PT, _LAST_R0)

    pltpu.async_copy(edges_hbm.at[NW + wid], didx, d0)

    @pl.loop(0, ROWS_PT, step=16)
    def _(i):
        zbuf[pl.ds(i, 16)] = jnp.zeros((16,), jnp.float32)

    @pl.loop(0, CHUNK, step=16)
    def _(i):
        ones_v[pl.ds(i, 16)] = jnp.full((16,), 1.0, jnp.float32)

    pltpu.sync_copy(zbuf, acc_sh.at[pl.ds(r0, ROWS_PT)])
    pltpu.make_async_copy(edges_hbm.at[NW + wid], didx, d0).wait()
    plsc.subcore_barrier()

    dsem = (d0, d1)

    def _dsc(j, b):
        pltpu.async_copy(ones_v, acc_sh.at[didx.at[pl.ds(j * CHUNK, CHUNK)]],
                         dsem[b], add=True)

    def _dscw(j, b):
        pltpu.make_async_copy(ones_v,
                              acc_sh.at[didx.at[pl.ds(j * CHUNK, CHUNK)]],
                              dsem[b]).wait()

    _dsc(0, 0)

    @pl.loop(0, (NCHUNK - 1) // 2)
    def _(blk):
        j0 = 1 + blk * 2
        _dsc(j0, 1)
        _dscw(j0 - 1, 0)
        _dsc(j0 + 1, 0)
        _dscw(j0, 1)

    _dscw(NCHUNK - 1, 0)
    plsc.subcore_barrier()
    pltpu.sync_copy(acc_sh.at[pl.ds(r0, ROWS_PT)], zbuf)
    pltpu.sync_copy(zbuf, out_hbm.at[pl.ds(cid * N_NODES + r0, ROWS_PT)])


@functools.partial(
    pl.kernel,
    out_type=jax.ShapeDtypeStruct((NC, N_NODES, D), jnp.float32),
    mesh=_MESH,
    scratch_types=[
        pltpu.VMEM((EPT,), jnp.int32),
        pltpu.VMEM((EPT,), jnp.int32),
        pltpu.VMEM((2, ACH, D), jnp.float32),
        pltpu.VMEM_SHARED((N_NODES, D), jnp.float32),
    ] + [pltpu.SemaphoreType.DMA] * 4,
)
def _agg_kernel(table_hbm, edges_hbm, out_hbm,
                sidx, didx, rowbuf, acc_sh, g0, g1, s0, s1):
    rows = (rowbuf.at[0], rowbuf.at[1])
    gsem = (g0, g1)
    ssem = (s0, s1)
    cid = lax.axis_index("c")
    sid = lax.axis_index("s")
    wid = cid * NS + sid
    r0 = jnp.minimum(sid * ROWS_PT, _LAST_R0)

    pltpu.async_copy(edges_hbm.at[wid], sidx, g0)
    pltpu.async_copy(edges_hbm.at[NW + wid], didx, g1)

    @pl.loop(0, ACH)
    def _(i):
        @pl.loop(0, D, step=16)
        def _(j):
            rowbuf[1, i, pl.ds(j, 16)] = jnp.zeros((16,), jnp.float32)

    def _gather(j, b):
        pltpu.async_copy(table_hbm.at[sidx.at[pl.ds(j * ACH, ACH)]],
                         rows[b], gsem[b])

    def _wait_gather(j, b):
        pltpu.make_async_copy(table_hbm.at[sidx.at[pl.ds(j * ACH, ACH)]],
                              rows[b], gsem[b]).wait()

    def _scatter(j, b):
        pltpu.async_copy(rows[b], acc_sh.at[didx.at[pl.ds(j * ACH, ACH)]],
                         ssem[b], add=True)

    def _wait_scatter(j, b):
        pltpu.make_async_copy(rows[b], acc_sh.at[didx.at[pl.ds(j * ACH, ACH)]],
                              ssem[b]).wait()

    def _step(j, b, wait_scat=True, issue_gather=True):
        _wait_gather(j, b)
        _scatter(j, b)
        if issue_gather:
            if wait_scat:
                _wait_scatter(j - 1, 1 - b)
            _gather(j + 1, 1 - b)

    pltpu.make_async_copy(edges_hbm.at[wid], sidx, g0).wait()
    _gather(0, 0)

    @pl.loop(0, DRAIN_C)
    def _(k):
        pltpu.sync_copy(rows[1], acc_sh.at[pl.ds(r0 + k * ACH, ACH)])

    pltpu.make_async_copy(edges_hbm.at[NW + wid], didx, g1).wait()
    plsc.subcore_barrier()

    _step(0, 0, wait_scat=False)
    _step(1, 1)
    _step(2, 0)

    @pl.loop(0, (ANCH - 5) // 2)
    def _(blk):
        j0 = 3 + blk * 2
        _step(j0, 1)
        _step(j0 + 1, 0)

    for j in range(3 + 2 * ((ANCH - 5) // 2), ANCH):
        _step(j, j % 2, issue_gather=(j < ANCH - 1))
    _wait_scatter(ANCH - 2, (ANCH - 2) % 2)
    _wait_scatter(ANCH - 1, (ANCH - 1) % 2)

    plsc.subcore_barrier()

    def _dfill(k):
        pltpu.async_copy(acc_sh.at[pl.ds(r0 + k * ACH, ACH)], rows[k % 2],
                         gsem[k % 2])

    def _dpush(k):
        pltpu.async_copy(rows[k % 2], out_hbm.at[cid, pl.ds(r0 + k * ACH, ACH)],
                         ssem[k % 2])

    _dfill(0)
    for k in range(DRAIN_C):
        pltpu.make_async_copy(acc_sh.at[pl.ds(r0 + k * ACH, ACH)],
                              rows[k % 2], gsem[k % 2]).wait()
        _dpush(k)
        if k + 1 < DRAIN_C:
            if k >= 1:
                pltpu.make_async_copy(
                    rows[(k - 1) % 2],
                    out_hbm.at[cid, pl.ds(r0 + (k - 1) * ACH, ACH)],
                    ssem[(k - 1) % 2]).wait()
            _dfill(k + 1)
    for k in (DRAIN_C - 2, DRAIN_C - 1):
        pltpu.make_async_copy(rows[k % 2],
                              out_hbm.at[cid, pl.ds(r0 + k * ACH, ACH)],
                              ssem[k % 2]).wait()


_RB = 2000
_GRID = N_NODES // _RB

_DN = (((1,), (0,)), ((), ()))


def _row_spec(width):
    return pl.BlockSpec((_RB, width), lambda i: (i, 0))


def _pair_spec(width):
    return pl.BlockSpec((NC, _RB, width), lambda i: (0, i, 0))


def _full_spec(shape):
    return pl.BlockSpec(shape, lambda i: tuple(0 for _ in shape))


def _prep_body(degp_ref, x_ref, xs_ref, dinv_ref, dinv2_ref):
    d = degp_ref[:, 0:1] + degp_ref[:, 1:2] + 1.0
    dc = lax.rsqrt(d)
    dinv_ref[...] = dc
    dinv2_ref[...] = 1.0 / d
    xs_ref[...] = x_ref[...] * dc


_prep = pl.pallas_call(
    _prep_body,
    grid=(_GRID,),
    in_specs=[pl.BlockSpec((_RB, NC), lambda i: (i, 0)), _row_spec(D)],
    out_specs=[_row_spec(D), _row_spec(1), _row_spec(1)],
    out_shape=[jax.ShapeDtypeStruct((N_NODES, D), jnp.float32),
               jax.ShapeDtypeStruct((N_NODES, 1), jnp.float32),
               jax.ShapeDtypeStruct((N_NODES, 1), jnp.float32)],
)


def _mid_body(pp_ref, x_ref, dinv_ref, dinv2_ref,
              w1_ref, b1_ref, w2_ref, y_ref, ys_ref):
    ax = (dinv_ref[...] * (pp_ref[0] + pp_ref[1])
          + dinv2_ref[...] * x_ref[...])
    h1 = lax.dot_general(ax, w1_ref[...], _DN,
                         preferred_element_type=jnp.float32)
    h1 = jnp.maximum(h1 + b1_ref[...], 0.0)
    y = lax.dot_general(h1, w2_ref[...], _DN,
                        preferred_element_type=jnp.float32)
    y_ref[...] = y
    ys_ref[...] = y * dinv_ref[...]


_mid = pl.pallas_call(
    _mid_body,
    grid=(_GRID,),
    in_specs=[_pair_spec(D), _row_spec(D), _row_spec(1),
              _row_spec(1), _full_spec((D, 2 * D)), _full_spec((1, 2 * D)),
              _full_spec((2 * D, D))],
    out_specs=[_row_spec(D), _row_spec(D)],
    out_shape=[jax.ShapeDtypeStruct((N_NODES, D), jnp.float32),
               jax.ShapeDtypeStruct((N_NODES, D), jnp.float32)],
)


def _final_body(qq_ref, y_ref, dinv_ref, dinv2_ref, b2_ref, out_ref):
    agg = dinv_ref[...] * (qq_ref[0] + qq_ref[1])
    out_ref[...] = jnp.maximum(agg + dinv2_ref[...] * y_ref[...]
                               + b2_ref[...], 0.0)


_final = pl.pallas_call(
    _final_body,
    grid=(_GRID,),
    in_specs=[_pair_spec(D), _row_spec(D), _row_spec(1),
              _row_spec(1), _full_spec((1, D))],
    out_specs=_row_spec(D),
    out_shape=jax.ShapeDtypeStruct((N_NODES, D), jnp.float32),
)


def kernel(x, edge_index, W1, b1, W2, b2):
    edges = edge_index.astype(jnp.int32).reshape(2 * NW, EPT)
    deg_p = _deg_kernel(edges).reshape(NC, N_NODES)
    xs, dinv, dinv2 = _prep(deg_p.T, x)
    agg1 = _agg_kernel(xs, edges)
    y, ys = _mid(agg1, x, dinv, dinv2, W1, b1.reshape(1, 2 * D), W2)
    agg2 = _agg_kernel(ys, edges)
    return _final(agg2, y, dinv, dinv2, b2.reshape(1, D))
